# baseline (speedup 1.0000x reference)
import functools
from math import sqrt

import jax
import jax.numpy as jnp
from jax import lax
from jax.experimental import pallas as pl
from jax.experimental.pallas import tpu as pltpu
from jax.experimental.pallas import tpu_sc as plsc

D_MODEL = 64
SCALE = sqrt(D_MODEL)

NC = 2
NS = 16
NW = NC * NS
LANES = 16

BLK = 128
ABUF = 4
NBUF = 4

_SC_PARAMS = pltpu.CompilerParams(
    use_tc_tiling_on_sc=True,
    needs_layout_passes=False,
)


def _mesh():
    return plsc.VectorSubcoreMesh(
        core_axis_name="c", subcore_axis_name="s", num_cores=NC, num_subcores=NS
    )


@functools.lru_cache(maxsize=None)
def _build_prep(V):
    n_full = V // BLK
    n_even = (n_full // NW) * NW
    per_w = n_even // NW
    assert per_w % ABUF == 0
    rem_full = n_full - n_even
    has_tail = V % BLK != 0

    tail_rows = (V % BLK) // 2

    @functools.partial(
        pl.kernel,
        out_type=jax.ShapeDtypeStruct((V // 2, 2 * D_MODEL), jnp.float32),
        mesh=_mesh(),
        scratch_types=[
            pltpu.VMEM((ABUF, D_MODEL, BLK), jnp.float32),
            pltpu.VMEM((ABUF, BLK // 2, 2 * D_MODEL), jnp.float32),
            pltpu.SemaphoreType.DMA,
            pltpu.SemaphoreType.DMA,
        ],
        compiler_params=_SC_PARAMS,
    )
    def prep_kernel(wt_hbm, tail_hbm, w2_hbm, slab_v, obuf_v, lsem, osem):
        wid = lax.axis_index("s") * NC + lax.axis_index("c")

        def load_slab(blk, buf, width):
            pltpu.async_copy(
                wt_hbm.at[:, pl.ds(blk * BLK, width)],
                slab_v.at[buf, :, pl.ds(0, width)],
                lsem,
            )

        def wait_slab(buf, width):
            pltpu.make_async_copy(
                wt_hbm.at[:, pl.ds(0, width)],
                slab_v.at[buf, :, pl.ds(0, width)],
                lsem,
            ).wait()

        def transpose_block(buf, width):
            @plsc.parallel_loop(0, width // 2, step=1, unroll=4)
            def _rows(q):
                for hc in range(2):
                    for j0 in range(D_MODEL // LANES):
                        col = plsc.load_gather(
                            slab_v.at[buf],
                            [
                                j0 * LANES + lax.iota(jnp.int32, LANES),
                                jnp.full((LANES,), 0, jnp.int32) + (2 * q + hc),
                            ],
                        )
                        obuf_v[buf, q, pl.ds(hc * D_MODEL + j0 * LANES, LANES)] = (
                            col * SCALE
                        )

        def store_block(blk, buf, width):
            pltpu.async_copy(
                obuf_v.at[buf, pl.ds(0, width // 2)],
                w2_hbm.at[pl.ds(blk * (BLK // 2), width // 2)],
                osem,
            )

        def wait_store(buf, width):
            pltpu.make_async_copy(
                obuf_v.at[buf, pl.ds(0, width // 2)],
                w2_hbm.at[pl.ds(0, width // 2)],
                osem,
            ).wait()

        for b in range(ABUF):
            load_slab(wid + NW * b, b, BLK)

        def outer(t, carry):
            for b in range(ABUF):
                k = t * ABUF + b
                wait_slab(b, BLK)

                @pl.when(t > 0)
                def _():
                    wait_store(b, BLK)

                transpose_block(b, BLK)
                store_block(wid + NW * k, b, BLK)

                @pl.when(k + ABUF < per_w)
                def _():
                    load_slab(wid + NW * (k + ABUF), b, BLK)

            return carry

        lax.fori_loop(0, per_w // ABUF, outer, 0)
        for b in range(ABUF):
            wait_store(b, BLK)

        @pl.when(wid < rem_full)
        def _rem_full():
            blk = n_even + wid
            load_slab(blk, 0, BLK)
            wait_slab(0, BLK)
            transpose_block(0, BLK)
            store_block(blk, 0, BLK)
            wait_store(0, BLK)

        if has_tail:
            @pl.when(wid == rem_full)
            def _rem_tail():
                pltpu.sync_copy(tail_hbm, obuf_v.at[0, pl.ds(0, tail_rows)])
                pltpu.sync_copy(
                    obuf_v.at[0, pl.ds(0, tail_rows)],
                    w2_hbm.at[pl.ds(n_full * (BLK // 2), tail_rows)],
                )

    return prep_kernel


@functools.lru_cache(maxsize=None)
def _build_lookup(SEQ, NBT, V):
    assert NBT == NW and SEQ % NBUF == 0

    @functools.partial(
        pl.kernel,
        out_type=jax.ShapeDtypeStruct((SEQ, 8, NBT, 8, 128), jnp.float32),
        mesh=_mesh(),
        scratch_types=[
            pltpu.VMEM((NBUF, 128), jnp.int32),
            pltpu.VMEM((NBUF, 128), jnp.int32),
            pltpu.VMEM((NBUF, 128, 2 * D_MODEL), jnp.float32),
            pltpu.VMEM((NBUF, 8, 8, 128), jnp.float32),
            pltpu.SemaphoreType.DMA,
            pltpu.SemaphoreType.DMA,
            pltpu.SemaphoreType.DMA,
        ],
        compiler_params=_SC_PARAMS,
    )
    def lookup_kernel(x_hbm, w2_hbm, out_hbm, idx_v, gidx_v, in_v, blk_v,
                      xsem, gsem, osem):
        wid = lax.axis_index("s") * NC + lax.axis_index("c")

        def start_chunk(s, b):
            pltpu.async_copy(x_hbm.at[s, wid], idx_v.at[b], xsem)

        def launch_gather(b):
            pltpu.make_async_copy(
                x_hbm.at[0, wid], idx_v.at[b], xsem
            ).wait()

            @plsc.parallel_loop(0, 128 // LANES, step=1, unroll=4)
            def _shift(v):
                sl = pl.ds(v * LANES, LANES)
                gidx_v[b, sl] = lax.shift_right_logical(idx_v[b, sl], 1)

            pltpu.async_copy(w2_hbm.at[gidx_v.at[b]], in_v.at[b], gsem)

        for b in range(NBUF):
            start_chunk(b, b)
        for b in range(NBUF):
            launch_gather(b)

        def outer(t, carry):
            for b in range(NBUF):
                s = t * NBUF + b
                pltpu.make_async_copy(
                    w2_hbm.at[gidx_v.at[b]], in_v.at[b], gsem
                ).wait()

                @pl.when(t > 0)
                def _():
                    pltpu.make_async_copy(
                        blk_v.at[b], out_hbm.at[0, :, wid], osem
                    ).wait()

                @plsc.parallel_loop(0, 8, step=1, unroll=2)
                def _grp(g):
                    lanes = g * LANES + lax.iota(jnp.int32, LANES)
                    hoff = (idx_v[b, pl.ds(g * LANES, LANES)] & 1) * D_MODEL
                    for j in range(D_MODEL):
                        val = plsc.load_gather(
                            in_v.at[b], [lanes, hoff + j]
                        )
                        blk_v[b, j // 8, j % 8, pl.ds(g * LANES, LANES)] = val

                pltpu.async_copy(
                    blk_v.at[b], out_hbm.at[s, :, wid], osem
                )

                @pl.when(s + NBUF < SEQ)
                def _():
                    start_chunk(s + NBUF, b)
                    launch_gather(b)

            return carry

        lax.fori_loop(0, SEQ // NBUF, outer, 0)
        for b in range(NBUF):
            pltpu.make_async_copy(
                blk_v.at[b], out_hbm.at[0, :, wid], osem
            ).wait()

    return lookup_kernel


def kernel(x, W):
    batch, seq = x.shape
    V = W.shape[0]
    wt = W.T
    n_tail = V % BLK
    tail = (W[V - n_tail:] * SCALE).reshape(n_tail // 2, 2 * D_MODEL)
    w2 = _build_prep(V)(wt, tail)
    xt = x.T.reshape(seq, batch // 128, 128).astype(jnp.int32)
    out5 = _build_lookup(seq, batch // 128, V)(xt, w2)
    return out5.transpose(2, 4, 0, 1, 3).reshape(batch, seq, D_MODEL)


# speedup vs baseline: 1.1599x; 1.1599x over previous
import functools
from math import sqrt

import jax
import jax.numpy as jnp
from jax import lax
from jax.experimental import pallas as pl
from jax.experimental.pallas import tpu as pltpu
from jax.experimental.pallas import tpu_sc as plsc

D_MODEL = 64
SCALE = sqrt(D_MODEL)

NC = 2
NS = 16
NW = NC * NS
LANES = 16

BLK = 128
ABUF = 2
NBUF = 4

_SC_PARAMS_TILED = pltpu.CompilerParams(
    use_tc_tiling_on_sc=True,
    needs_layout_passes=False,
)
_SC_PARAMS_LINEAR = pltpu.CompilerParams(
    use_tc_tiling_on_sc=False,
    needs_layout_passes=False,
)


def _mesh():
    return plsc.VectorSubcoreMesh(
        core_axis_name="c", subcore_axis_name="s", num_cores=NC, num_subcores=NS
    )


@functools.lru_cache(maxsize=None)
def _build_prep(V):
    n_full = V // BLK
    n_even = (n_full // NW) * NW
    per_w = n_even // NW
    assert per_w % ABUF == 0
    rem_full = n_full - n_even
    has_tail = V % BLK != 0

    tail_rows = (V % BLK) // 2

    @functools.partial(
        pl.kernel,
        out_type=jax.ShapeDtypeStruct((V // 2, 2 * D_MODEL), jnp.float32),
        mesh=_mesh(),
        scratch_types=[
            pltpu.VMEM((ABUF, D_MODEL, BLK + 8), jnp.float32),
            pltpu.VMEM((ABUF, BLK // 2, 2 * D_MODEL), jnp.float32),
            pltpu.SemaphoreType.DMA,
            pltpu.SemaphoreType.DMA,
        ],
        compiler_params=_SC_PARAMS_TILED,
    )
    def prep_kernel(wt_hbm, tail_hbm, w2_hbm, slab_v, obuf_v, lsem, osem):
        wid = lax.axis_index("s") * NC + lax.axis_index("c")

        def load_slab(blk, buf, width):
            pltpu.async_copy(
                wt_hbm.at[:, pl.ds(blk * BLK, width)],
                slab_v.at[buf, :, pl.ds(0, width)],
                lsem,
            )

        def wait_slab(buf, width):
            pltpu.make_async_copy(
                wt_hbm.at[:, pl.ds(0, width)],
                slab_v.at[buf, :, pl.ds(0, width)],
                lsem,
            ).wait()

        def transpose_block(buf, width):
            @plsc.parallel_loop(0, width // 2, step=1, unroll=4)
            def _rows(q):
                for hc in range(2):
                    for j0 in range(D_MODEL // LANES):
                        col = plsc.load_gather(
                            slab_v.at[buf],
                            [
                                j0 * LANES + lax.iota(jnp.int32, LANES),
                                jnp.full((LANES,), 0, jnp.int32) + (2 * q + hc),
                            ],
                        )
                        obuf_v[buf, q, pl.ds(hc * D_MODEL + j0 * LANES, LANES)] = (
                            col * SCALE
                        )

        def store_block(blk, buf, width):
            pltpu.async_copy(
                obuf_v.at[buf, pl.ds(0, width // 2)],
                w2_hbm.at[pl.ds(blk * (BLK // 2), width // 2)],
                osem,
            )

        def wait_store(buf, width):
            pltpu.make_async_copy(
                obuf_v.at[buf, pl.ds(0, width // 2)],
                w2_hbm.at[pl.ds(0, width // 2)],
                osem,
            ).wait()

        for b in range(ABUF):
            load_slab(wid + NW * b, b, BLK)

        def outer(t, carry):
            for b in range(ABUF):
                k = t * ABUF + b
                wait_slab(b, BLK)

                @pl.when(t > 0)
                def _():
                    wait_store(b, BLK)

                transpose_block(b, BLK)
                store_block(wid + NW * k, b, BLK)

                @pl.when(k + ABUF < per_w)
                def _():
                    load_slab(wid + NW * (k + ABUF), b, BLK)

            return carry

        lax.fori_loop(0, per_w // ABUF, outer, 0)
        for b in range(ABUF):
            wait_store(b, BLK)

        @pl.when(wid < rem_full)
        def _rem_full():
            blk = n_even + wid
            load_slab(blk, 0, BLK)
            wait_slab(0, BLK)
            transpose_block(0, BLK)
            store_block(blk, 0, BLK)
            wait_store(0, BLK)

        if has_tail:
            @pl.when(wid == rem_full)
            def _rem_tail():
                pltpu.sync_copy(tail_hbm, obuf_v.at[0, pl.ds(0, tail_rows)])
                pltpu.sync_copy(
                    obuf_v.at[0, pl.ds(0, tail_rows)],
                    w2_hbm.at[pl.ds(n_full * (BLK // 2), tail_rows)],
                )

    return prep_kernel


@functools.lru_cache(maxsize=None)
def _build_lookup(SEQ, NBT, V):
    assert NBT == NW and SEQ % NBUF == 0

    @functools.partial(
        pl.kernel,
        out_type=jax.ShapeDtypeStruct((SEQ, 8, NBT, 8, 128), jnp.float32),
        mesh=_mesh(),
        scratch_types=[
            pltpu.VMEM((NBUF, 128), jnp.int32),
            pltpu.VMEM((NBUF, 128), jnp.int32),
            pltpu.VMEM((NBUF, 128, 2 * D_MODEL), jnp.float32),
            pltpu.VMEM((128, 2 * D_MODEL + 8), jnp.float32),
            pltpu.VMEM((NBUF, 8, 8, 128), jnp.float32),
            pltpu.SemaphoreType.DMA,
            pltpu.SemaphoreType.DMA,
            pltpu.SemaphoreType.DMA,
        ],
        compiler_params=_SC_PARAMS_LINEAR,
    )
    def lookup_kernel(x_hbm, w2_hbm, out_hbm, idx_v, gidx_v, raw_v, in_v, blk_v,
                      xsem, gsem, osem):
        wid = lax.axis_index("s") * NC + lax.axis_index("c")

        def start_chunk(s, b):
            pltpu.async_copy(x_hbm.at[s, wid], idx_v.at[b], xsem)

        def launch_gather(b):
            pltpu.make_async_copy(
                x_hbm.at[0, wid], idx_v.at[b], xsem
            ).wait()

            @plsc.parallel_loop(0, 128 // LANES, step=1, unroll=4)
            def _shift(v):
                sl = pl.ds(v * LANES, LANES)
                gidx_v[b, sl] = lax.shift_right_logical(idx_v[b, sl], 1)

            pltpu.async_copy(w2_hbm.at[gidx_v.at[b]], raw_v.at[b], gsem)

        for b in range(NBUF):
            start_chunk(b, b)
        for b in range(NBUF):
            launch_gather(b)

        def outer(t, carry):
            for b in range(NBUF):
                s = t * NBUF + b
                pltpu.make_async_copy(
                    w2_hbm.at[gidx_v.at[b]], raw_v.at[b], gsem
                ).wait()
                @plsc.parallel_loop(0, 128, step=1, unroll=4)
                def _repad(l):
                    for c in range(2 * D_MODEL // LANES):
                        sl = pl.ds(c * LANES, LANES)
                        in_v[l, sl] = raw_v[b, l, sl]

                @pl.when(t > 0)
                def _():
                    pltpu.make_async_copy(
                        blk_v.at[b], out_hbm.at[0, :, wid], osem
                    ).wait()

                @plsc.parallel_loop(0, 8, step=1, unroll=2)
                def _grp(g):
                    lanes = g * LANES + lax.iota(jnp.int32, LANES)
                    hoff = (idx_v[b, pl.ds(g * LANES, LANES)] & 1) * D_MODEL
                    for j in range(D_MODEL):
                        val = plsc.load_gather(in_v, [lanes, hoff + j])
                        blk_v[b, j // 8, j % 8, pl.ds(g * LANES, LANES)] = val

                pltpu.async_copy(
                    blk_v.at[b], out_hbm.at[s, :, wid], osem
                )

                @pl.when(s + NBUF < SEQ)
                def _():
                    start_chunk(s + NBUF, b)
                    launch_gather(b)

            return carry

        lax.fori_loop(0, SEQ // NBUF, outer, 0)
        for b in range(NBUF):
            pltpu.make_async_copy(
                blk_v.at[b], out_hbm.at[0, :, wid], osem
            ).wait()

    return lookup_kernel


def kernel(x, W):
    batch, seq = x.shape
    V = W.shape[0]
    wt = W.T
    n_tail = V % BLK
    tail = (W[V - n_tail:] * SCALE).reshape(n_tail // 2, 2 * D_MODEL)
    w2 = _build_prep(V)(wt, tail)
    xt = x.T.reshape(seq, batch // 128, 128).astype(jnp.int32)
    out5 = _build_lookup(seq, batch // 128, V)(xt, w2)
    return out5.transpose(2, 4, 0, 1, 3).reshape(batch, seq, D_MODEL)


# speedup vs baseline: 1.4143x; 1.2193x over previous
import functools
from math import sqrt

import jax
import jax.numpy as jnp
from jax import lax
from jax.experimental import pallas as pl
from jax.experimental.pallas import tpu as pltpu
from jax.experimental.pallas import tpu_sc as plsc

D_MODEL = 64
SCALE = sqrt(D_MODEL)

NC = 2
NS = 16
NW = NC * NS
LANES = 16

CHUNK = 128
NBUF = 4


@functools.lru_cache(maxsize=None)
def _build(B, V):
    assert B % (NW * CHUNK * NBUF) == 0
    b_per_w = B // NW
    n_chunks = b_per_w // CHUNK
    n_outer = n_chunks // NBUF
    mesh = plsc.VectorSubcoreMesh(
        core_axis_name="c", subcore_axis_name="s", num_cores=NC, num_subcores=NS
    )

    @functools.partial(
        pl.kernel,
        out_type=jax.ShapeDtypeStruct((B, D_MODEL), jnp.float32),
        mesh=mesh,
        scratch_types=[
            pltpu.VMEM((n_chunks, CHUNK), jnp.int32),
            pltpu.VMEM((NBUF, CHUNK, D_MODEL), jnp.float32),
            pltpu.VMEM((NBUF, CHUNK, D_MODEL), jnp.float32),
            pltpu.SemaphoreType.DMA,
            pltpu.SemaphoreType.DMA,
        ],
        compiler_params=pltpu.CompilerParams(
            use_tc_tiling_on_sc=False,
            skip_device_barrier=True,
        ),
    )
    def emb_kernel(x_hbm, w_hbm, out_hbm, idx_v, in_v, out_v, gsem, osem):
        wid = lax.axis_index("s") * NC + lax.axis_index("c")
        base = wid * b_per_w
        pltpu.sync_copy(x_hbm.at[wid], idx_v)

        for b in range(NBUF):
            pltpu.async_copy(w_hbm.at[idx_v.at[b]], in_v.at[b], gsem)

        def outer(t, carry):
            for b in range(NBUF):
                g = t * NBUF + b
                pltpu.make_async_copy(
                    w_hbm.at[idx_v.at[b]], in_v.at[b], gsem
                ).wait()

                @pl.when(t > 0)
                def _wait_out():
                    pltpu.make_async_copy(
                        out_v.at[b], out_hbm.at[pl.ds(base, CHUNK)], osem
                    ).wait()

                @plsc.parallel_loop(0, CHUNK, step=1, unroll=8)
                def _scale(i):
                    for j in range(D_MODEL // LANES):
                        sl = pl.ds(j * LANES, LANES)
                        out_v[b, i, sl] = in_v[b, i, sl] * SCALE

                pltpu.async_copy(
                    out_v.at[b], out_hbm.at[pl.ds(base + g * CHUNK, CHUNK)], osem
                )

                @pl.when(g + NBUF < n_chunks)
                def _next_gather():
                    pltpu.async_copy(
                        w_hbm.at[idx_v.at[g + NBUF]], in_v.at[b], gsem
                    )

            return carry

        lax.fori_loop(0, n_outer, outer, 0)

        for b in range(NBUF):
            pltpu.make_async_copy(
                out_v.at[b], out_hbm.at[pl.ds(base, CHUNK)], osem
            ).wait()

    return emb_kernel


def kernel(x, W):
    batch, seq = x.shape
    B = batch * seq
    x_flat = x.reshape(NW, B // NW // CHUNK, CHUNK).astype(jnp.int32)
    out = _build(B, W.shape[0])(x_flat, W)
    return out.reshape(batch, seq, D_MODEL)
